# baseline (speedup 1.0000x reference)
import functools

import jax
import jax.numpy as jnp
from jax import lax
from jax.experimental import pallas as pl
from jax.experimental.pallas import tpu as pltpu
from jax.experimental.pallas import tpu_sc as plsc

_TN = 512
_PTS_PER_CHUNK = 32


def _p1t_body(p1_ref, w1_ref, b1r_ref, out_ref):
    dn = (((0,), (1,)), ((), ()))
    out_ref[...] = lax.dot_general(p1_ref[0], w1_ref[...], dn,
                                   preferred_element_type=jnp.float32) + b1r_ref[...]


def _knn_body(xyz1_ref, xyz2_ref, idx_ref, w_ref, *, m, k_nn, batch0):
    x1 = xyz1_ref[0]
    x2 = xyz2_ref[0].T

    d = None
    for c in range(x1.shape[0]):
        diff = x2[:, c:c + 1] - x1[c:c + 1, :]
        sq = diff * diff
        d = sq if d is None else d + sq

    iota = lax.broadcasted_iota(jnp.int32, d.shape, 1)
    big = jnp.float32(3.0e38)

    d_ks, i_ks = [], []
    dd = d
    for _ in range(k_nn):
        dmin = jnp.min(dd, axis=1, keepdims=True)
        sel = dd == dmin
        idx = jnp.min(jnp.where(sel, iota, jnp.int32(m)), axis=1,
                      keepdims=True)
        d_ks.append(dmin)
        i_ks.append(idx)
        dd = jnp.where(iota == idx, big, dd)

    recips = [1.0 / (dk + jnp.float32(1e-8)) for dk in d_ks]
    norm = functools.reduce(lambda a, b: a + b, recips)
    weights = [r / norm for r in recips]

    base = (batch0 + pl.program_id(0)) * m
    idx_ref[0] = jnp.concatenate(i_ks, axis=1) + base
    w_ref[0] = jnp.concatenate(
        [jnp.broadcast_to(wk, (wk.shape[0], 16)) for wk in weights], axis=1)


def _interp_tc_body(p2_ref, w2_ref, b2_ref, interp_ref, out_ref):
    dn = (((1,), (0,)), ((), ()))
    p2 = lax.dot_general(w2_ref[...], p2_ref[0], dn,
                         preferred_element_type=jnp.float32) + b2_ref[...]
    out_ref[0] = p2 + interp_ref[...].T


def _interp_tc_body2(p2_ref, w2_ref, b2_ref, interp_ref, prev_ref, out_ref):
    del prev_ref
    _interp_tc_body(p2_ref, w2_ref, b2_ref, interp_ref, out_ref)


def _sc_gather(p1t_flat, gidx, gw, n_rows, c_out):
    info = plsc.get_sparse_core_info()
    nc, ns = info.num_cores, info.num_subcores
    nw = nc * ns
    ppc = _PTS_PER_CHUNK
    rows_per_chunk = ppc * 3
    chunks_total = n_rows // ppc
    chunks_per_w = chunks_total // nw
    mesh = plsc.VectorSubcoreMesh(core_axis_name="c", subcore_axis_name="s")
    cl = c_out // 16

    @functools.partial(
        pl.kernel, mesh=mesh,
        out_type=jax.ShapeDtypeStruct((n_rows, c_out), jnp.float32),
        scratch_types=[
            pltpu.VMEM((chunks_per_w, rows_per_chunk), jnp.int32),
            pltpu.VMEM((chunks_per_w * rows_per_chunk // 8, 128), jnp.float32),
            pltpu.VMEM((rows_per_chunk, c_out), jnp.float32),
            pltpu.VMEM((rows_per_chunk, c_out), jnp.float32),
            pltpu.VMEM((ppc, c_out), jnp.float32),
            pltpu.VMEM((ppc, c_out), jnp.float32),
            pltpu.SemaphoreType.DMA,
            pltpu.SemaphoreType.DMA,
            pltpu.SemaphoreType.DMA,
            pltpu.SemaphoreType.DMA,
        ],
    )
    def k(p1t_hbm, gidx_hbm, gw_hbm, out_hbm,
          idx_all, w_all, rows_a, rows_b, out_a, out_b, gs_a, gs_b, os_a, os_b):
        wid = lax.axis_index("s") * nc + lax.axis_index("c")
        rows_v = (rows_a, rows_b)
        out_v = (out_a, out_b)
        gsem = (gs_a, gs_b)
        osem = (os_a, os_b)

        w_rows = chunks_per_w * rows_per_chunk // 8
        pltpu.sync_copy(gidx_hbm.at[pl.ds(wid * chunks_per_w, chunks_per_w)],
                        idx_all)
        pltpu.sync_copy(gw_hbm.at[pl.ds(wid * w_rows, w_rows)], w_all)

        pltpu.async_copy(p1t_hbm.at[idx_all.at[0]], rows_v[0], gsem[0])
        pltpu.async_copy(p1t_hbm.at[idx_all.at[1]], rows_v[1], gsem[1])

        @pl.loop(0, chunks_per_w, step=2)
        def chunk_loop(g):
            for bsel in range(2):
                j = g + bsel
                rv = rows_v[bsel]
                ov = out_v[bsel]
                pltpu.make_async_copy(p1t_hbm.at[idx_all.at[j]], rv,
                                      gsem[bsel]).wait()

                @pl.when(g > 0)
                def _(ov=ov, bsel=bsel):
                    pltpu.make_async_copy(
                        ov, out_hbm.at[pl.ds(0, ppc)], osem[bsel]).wait()

                wbase = j * rows_per_chunk

                @plsc.parallel_loop(0, ppc, unroll=2)
                def point_body(p, rv=rv, ov=ov, wbase=wbase):
                    q = p * 3
                    wk = []
                    for k0 in range(3):
                        f = wbase + q + k0
                        wk.append(w_all[f // 8, pl.ds((f % 8) * 16, 16)])
                    for c in range(cl):
                        s = pl.ds(c * 16, 16)
                        acc = (rv[q, s] * wk[0]
                               + rv[q + 1, s] * wk[1]
                               + rv[q + 2, s] * wk[2])
                        ov[p, s] = acc

                pltpu.async_copy(
                    ov, out_hbm.at[pl.ds((wid * chunks_per_w + j) * ppc, ppc)],
                    osem[bsel])

                @pl.when(j + 2 < chunks_per_w)
                def _(rv=rv, bsel=bsel, j=j):
                    pltpu.async_copy(p1t_hbm.at[idx_all.at[j + 2]], rv,
                                     gsem[bsel])

        for bsel in range(2):
            pltpu.make_async_copy(out_v[bsel], out_hbm.at[pl.ds(0, ppc)],
                                  osem[bsel]).wait()

    return k(p1t_flat, gidx, gw)


def kernel(xyz_1, xyz_2, points_1, points_2, W1, b1, W2, b2):
    b, _, m = xyz_1.shape
    n = xyz_2.shape[2]
    c_in = points_1.shape[1]
    c_out = points_2.shape[1]
    tn = min(_TN, n)
    nt = n // tn
    nh = 2 if b % 2 == 0 else 1
    bh = b // nh

    b1r = b1[None, :]
    b2c = b2[:, None]

    p1t_flat = pl.pallas_call(
        _p1t_body,
        grid=(b,),
        in_specs=[
            pl.BlockSpec((1, c_in, m), lambda bi: (bi, 0, 0)),
            pl.BlockSpec((c_out, c_in), lambda bi: (0, 0)),
            pl.BlockSpec((1, c_out), lambda bi: (0, 0)),
        ],
        out_specs=pl.BlockSpec((m, c_out), lambda bi: (bi, 0)),
        out_shape=jax.ShapeDtypeStruct((b * m, c_out), jnp.float32),
    )(points_1, W1, b1r)

    out = None
    for h in range(nh):
        b0 = h * bh
        idx3, w3 = pl.pallas_call(
            functools.partial(_knn_body, m=m, k_nn=3, batch0=b0),
            grid=(bh, nt),
            in_specs=[
                pl.BlockSpec((1, 3, m), lambda bi, ti, b0=b0: (bi + b0, 0, 0)),
                pl.BlockSpec((1, 3, tn), lambda bi, ti, b0=b0: (bi + b0, 0, ti)),
            ],
            out_specs=[
                pl.BlockSpec((1, tn, 3), lambda bi, ti: (bi, ti, 0)),
                pl.BlockSpec((1, tn, 48), lambda bi, ti: (bi, ti, 0)),
            ],
            out_shape=[
                jax.ShapeDtypeStruct((bh, n, 3), jnp.int32),
                jax.ShapeDtypeStruct((bh, n, 48), jnp.float32),
            ],
        )(xyz_1, xyz_2)

        n_rows = bh * n
        gidx = idx3.reshape(n_rows // _PTS_PER_CHUNK, _PTS_PER_CHUNK * 3)
        gw = w3.reshape(n_rows * 3 // 8, 128)

        interp = _sc_gather(p1t_flat, gidx, gw, n_rows, c_out)

        in_specs = [
            pl.BlockSpec((1, c_out, tn), lambda bi, ti, b0=b0: (bi + b0, 0, ti)),
            pl.BlockSpec((c_out, c_out), lambda bi, ti: (0, 0)),
            pl.BlockSpec((c_out, 1), lambda bi, ti: (0, 0)),
            pl.BlockSpec((tn, c_out), lambda bi, ti: (bi * nt + ti, 0)),
        ]
        out_spec = pl.BlockSpec((1, c_out, tn),
                                lambda bi, ti, b0=b0: (bi + b0, 0, ti))
        out_shape = jax.ShapeDtypeStruct((b, c_out, n), jnp.float32)
        if h == 0:
            out = pl.pallas_call(
                _interp_tc_body, grid=(bh, nt), in_specs=in_specs,
                out_specs=out_spec, out_shape=out_shape,
            )(points_2, W2, b2c, interp)
        else:
            out = pl.pallas_call(
                _interp_tc_body2, grid=(bh, nt),
                in_specs=in_specs + [pl.BlockSpec(memory_space=pl.ANY)],
                out_specs=out_spec, out_shape=out_shape,
                input_output_aliases={4: 0},
            )(points_2, W2, b2c, interp, out)
    return (xyz_2, out)


# speedup vs baseline: 1.0542x; 1.0542x over previous
import functools

import jax
import jax.numpy as jnp
from jax import lax
from jax.experimental import pallas as pl
from jax.experimental.pallas import tpu as pltpu
from jax.experimental.pallas import tpu_sc as plsc

_TN = 1024
_PTS_PER_CHUNK = 32


def _p1t_body(p1_ref, w1_ref, b1r_ref, out_ref):
    dn = (((0,), (1,)), ((), ()))
    out_ref[...] = lax.dot_general(p1_ref[0], w1_ref[...], dn,
                                   preferred_element_type=jnp.float32) + b1r_ref[...]


def _knn_body(xyz1_ref, xyz2_ref, idx_ref, w_ref, *, m, k_nn, batch0):
    x1 = xyz1_ref[0]
    x2 = xyz2_ref[0].T

    d = None
    for c in range(x1.shape[0]):
        diff = x2[:, c:c + 1] - x1[c:c + 1, :]
        sq = diff * diff
        d = sq if d is None else d + sq

    iota = lax.broadcasted_iota(jnp.int32, d.shape, 1)
    big = jnp.float32(3.0e38)

    d_ks, i_ks = [], []
    dd = d
    for _ in range(k_nn):
        dmin = jnp.min(dd, axis=1, keepdims=True)
        sel = dd == dmin
        idx = jnp.min(jnp.where(sel, iota, jnp.int32(m)), axis=1,
                      keepdims=True)
        d_ks.append(dmin)
        i_ks.append(idx)
        dd = jnp.where(iota == idx, big, dd)

    recips = [1.0 / (dk + jnp.float32(1e-8)) for dk in d_ks]
    norm = functools.reduce(lambda a, b: a + b, recips)
    weights = [r / norm for r in recips]

    base = (batch0 + pl.program_id(0)) * m
    idx_ref[0] = jnp.concatenate(i_ks, axis=1) + base
    w_ref[0] = jnp.concatenate(
        [jnp.broadcast_to(wk, (wk.shape[0], 16)) for wk in weights], axis=1)


def _interp_tc_body(p2_ref, w2_ref, b2_ref, interp_ref, out_ref):
    dn = (((1,), (0,)), ((), ()))
    p2 = lax.dot_general(w2_ref[...], p2_ref[0], dn,
                         preferred_element_type=jnp.float32) + b2_ref[...]
    out_ref[0] = p2 + interp_ref[...].T


def _interp_tc_body2(p2_ref, w2_ref, b2_ref, interp_ref, prev_ref, out_ref):
    del prev_ref
    _interp_tc_body(p2_ref, w2_ref, b2_ref, interp_ref, out_ref)


def _sc_gather(p1t_flat, gidx, gw, n_rows, c_out):
    info = plsc.get_sparse_core_info()
    nc, ns = info.num_cores, info.num_subcores
    nw = nc * ns
    ppc = _PTS_PER_CHUNK
    rows_per_chunk = ppc * 3
    chunks_total = n_rows // ppc
    chunks_per_w = chunks_total // nw
    mesh = plsc.VectorSubcoreMesh(core_axis_name="c", subcore_axis_name="s")
    cl = c_out // 16

    @functools.partial(
        pl.kernel, mesh=mesh,
        out_type=jax.ShapeDtypeStruct((n_rows, c_out), jnp.float32),
        scratch_types=[
            pltpu.VMEM((chunks_per_w, rows_per_chunk), jnp.int32),
            pltpu.VMEM((chunks_per_w * rows_per_chunk // 8, 128), jnp.float32),
            pltpu.VMEM((rows_per_chunk, c_out), jnp.float32),
            pltpu.VMEM((rows_per_chunk, c_out), jnp.float32),
            pltpu.VMEM((ppc, c_out), jnp.float32),
            pltpu.VMEM((ppc, c_out), jnp.float32),
            pltpu.SemaphoreType.DMA,
            pltpu.SemaphoreType.DMA,
            pltpu.SemaphoreType.DMA,
            pltpu.SemaphoreType.DMA,
        ],
    )
    def k(p1t_hbm, gidx_hbm, gw_hbm, out_hbm,
          idx_all, w_all, rows_a, rows_b, out_a, out_b, gs_a, gs_b, os_a, os_b):
        wid = lax.axis_index("s") * nc + lax.axis_index("c")
        rows_v = (rows_a, rows_b)
        out_v = (out_a, out_b)
        gsem = (gs_a, gs_b)
        osem = (os_a, os_b)

        w_rows = chunks_per_w * rows_per_chunk // 8
        pltpu.sync_copy(gidx_hbm.at[pl.ds(wid * chunks_per_w, chunks_per_w)],
                        idx_all)
        pltpu.sync_copy(gw_hbm.at[pl.ds(wid * w_rows, w_rows)], w_all)

        pltpu.async_copy(p1t_hbm.at[idx_all.at[0]], rows_v[0], gsem[0])
        pltpu.async_copy(p1t_hbm.at[idx_all.at[1]], rows_v[1], gsem[1])

        @pl.loop(0, chunks_per_w, step=2)
        def chunk_loop(g):
            for bsel in range(2):
                j = g + bsel
                rv = rows_v[bsel]
                ov = out_v[bsel]
                pltpu.make_async_copy(p1t_hbm.at[idx_all.at[j]], rv,
                                      gsem[bsel]).wait()

                @pl.when(g > 0)
                def _(ov=ov, bsel=bsel):
                    pltpu.make_async_copy(
                        ov, out_hbm.at[pl.ds(0, ppc)], osem[bsel]).wait()

                wbase = j * rows_per_chunk

                @plsc.parallel_loop(0, ppc, unroll=2)
                def point_body(p, rv=rv, ov=ov, wbase=wbase):
                    q = p * 3
                    wk = []
                    for k0 in range(3):
                        f = wbase + q + k0
                        wk.append(w_all[f // 8, pl.ds((f % 8) * 16, 16)])
                    for c in range(cl):
                        s = pl.ds(c * 16, 16)
                        acc = (rv[q, s] * wk[0]
                               + rv[q + 1, s] * wk[1]
                               + rv[q + 2, s] * wk[2])
                        ov[p, s] = acc

                pltpu.async_copy(
                    ov, out_hbm.at[pl.ds((wid * chunks_per_w + j) * ppc, ppc)],
                    osem[bsel])

                @pl.when(j + 2 < chunks_per_w)
                def _(rv=rv, bsel=bsel, j=j):
                    pltpu.async_copy(p1t_hbm.at[idx_all.at[j + 2]], rv,
                                     gsem[bsel])

        for bsel in range(2):
            pltpu.make_async_copy(out_v[bsel], out_hbm.at[pl.ds(0, ppc)],
                                  osem[bsel]).wait()

    return k(p1t_flat, gidx, gw)


def kernel(xyz_1, xyz_2, points_1, points_2, W1, b1, W2, b2):
    b, _, m = xyz_1.shape
    n = xyz_2.shape[2]
    c_in = points_1.shape[1]
    c_out = points_2.shape[1]
    tn = min(_TN, n)
    nt = n // tn
    nh = 2 if b % 2 == 0 else 1
    bh = b // nh

    b1r = b1[None, :]
    b2c = b2[:, None]

    p1t_flat = pl.pallas_call(
        _p1t_body,
        grid=(b,),
        in_specs=[
            pl.BlockSpec((1, c_in, m), lambda bi: (bi, 0, 0)),
            pl.BlockSpec((c_out, c_in), lambda bi: (0, 0)),
            pl.BlockSpec((1, c_out), lambda bi: (0, 0)),
        ],
        out_specs=pl.BlockSpec((m, c_out), lambda bi: (bi, 0)),
        out_shape=jax.ShapeDtypeStruct((b * m, c_out), jnp.float32),
    )(points_1, W1, b1r)

    out = None
    for h in range(nh):
        b0 = h * bh
        idx3, w3 = pl.pallas_call(
            functools.partial(_knn_body, m=m, k_nn=3, batch0=b0),
            grid=(bh, nt),
            in_specs=[
                pl.BlockSpec((1, 3, m), lambda bi, ti, b0=b0: (bi + b0, 0, 0)),
                pl.BlockSpec((1, 3, tn), lambda bi, ti, b0=b0: (bi + b0, 0, ti)),
            ],
            out_specs=[
                pl.BlockSpec((1, tn, 3), lambda bi, ti: (bi, ti, 0)),
                pl.BlockSpec((1, tn, 48), lambda bi, ti: (bi, ti, 0)),
            ],
            out_shape=[
                jax.ShapeDtypeStruct((bh, n, 3), jnp.int32),
                jax.ShapeDtypeStruct((bh, n, 48), jnp.float32),
            ],
        )(xyz_1, xyz_2)

        n_rows = bh * n
        gidx = idx3.reshape(n_rows // _PTS_PER_CHUNK, _PTS_PER_CHUNK * 3)
        gw = w3.reshape(n_rows * 3 // 8, 128)

        interp = _sc_gather(p1t_flat, gidx, gw, n_rows, c_out)

        in_specs = [
            pl.BlockSpec((1, c_out, tn), lambda bi, ti, b0=b0: (bi + b0, 0, ti)),
            pl.BlockSpec((c_out, c_out), lambda bi, ti: (0, 0)),
            pl.BlockSpec((c_out, 1), lambda bi, ti: (0, 0)),
            pl.BlockSpec((tn, c_out), lambda bi, ti: (bi * nt + ti, 0)),
        ]
        out_spec = pl.BlockSpec((1, c_out, tn),
                                lambda bi, ti, b0=b0: (bi + b0, 0, ti))
        out_shape = jax.ShapeDtypeStruct((b, c_out, n), jnp.float32)
        if h == 0:
            out = pl.pallas_call(
                _interp_tc_body, grid=(bh, nt), in_specs=in_specs,
                out_specs=out_spec, out_shape=out_shape,
            )(points_2, W2, b2c, interp)
        else:
            out = pl.pallas_call(
                _interp_tc_body2, grid=(bh, nt),
                in_specs=in_specs + [pl.BlockSpec(memory_space=pl.ANY)],
                out_specs=out_spec, out_shape=out_shape,
                input_output_aliases={4: 0},
            )(points_2, W2, b2c, interp, out)
    return (xyz_2, out)


# speedup vs baseline: 1.0753x; 1.0201x over previous
import functools

import jax
import jax.numpy as jnp
from jax import lax
from jax.experimental import pallas as pl
from jax.experimental.pallas import tpu as pltpu
from jax.experimental.pallas import tpu_sc as plsc

_TN = 2048
_PTS_PER_CHUNK = 32


def _p1t_body(p1_ref, w1_ref, b1r_ref, out_ref):
    dn = (((0,), (1,)), ((), ()))
    out_ref[...] = lax.dot_general(p1_ref[0], w1_ref[...], dn,
                                   preferred_element_type=jnp.float32) + b1r_ref[...]


def _knn_body(xyz1_ref, xyz2_ref, idx_ref, w_ref, *, m, k_nn, batch0):
    x1 = xyz1_ref[0]
    x2 = xyz2_ref[0].T

    d = None
    for c in range(x1.shape[0]):
        diff = x2[:, c:c + 1] - x1[c:c + 1, :]
        sq = diff * diff
        d = sq if d is None else d + sq

    iota = lax.broadcasted_iota(jnp.int32, d.shape, 1)
    big = jnp.float32(3.0e38)

    d_ks, i_ks = [], []
    dd = d
    for _ in range(k_nn):
        dmin = jnp.min(dd, axis=1, keepdims=True)
        sel = dd == dmin
        idx = jnp.min(jnp.where(sel, iota, jnp.int32(m)), axis=1,
                      keepdims=True)
        d_ks.append(dmin)
        i_ks.append(idx)
        dd = jnp.where(iota == idx, big, dd)

    recips = [1.0 / (dk + jnp.float32(1e-8)) for dk in d_ks]
    norm = functools.reduce(lambda a, b: a + b, recips)
    weights = [r / norm for r in recips]

    base = (batch0 + pl.program_id(0)) * m
    idx_ref[0] = jnp.concatenate(i_ks, axis=1) + base
    w_ref[0] = jnp.concatenate(
        [jnp.broadcast_to(wk, (wk.shape[0], 16)) for wk in weights], axis=1)


def _interp_tc_body(p2_ref, w2_ref, b2_ref, interp_ref, out_ref):
    dn = (((1,), (0,)), ((), ()))
    p2 = lax.dot_general(w2_ref[...], p2_ref[0], dn,
                         preferred_element_type=jnp.float32) + b2_ref[...]
    out_ref[0] = p2 + interp_ref[...].T


def _interp_tc_body2(p2_ref, w2_ref, b2_ref, interp_ref, prev_ref, out_ref):
    del prev_ref
    _interp_tc_body(p2_ref, w2_ref, b2_ref, interp_ref, out_ref)


def _sc_gather(p1t_flat, gidx, gw, n_rows, c_out):
    info = plsc.get_sparse_core_info()
    nc, ns = info.num_cores, info.num_subcores
    nw = nc * ns
    ppc = _PTS_PER_CHUNK
    rows_per_chunk = ppc * 3
    chunks_total = n_rows // ppc
    chunks_per_w = chunks_total // nw
    mesh = plsc.VectorSubcoreMesh(core_axis_name="c", subcore_axis_name="s")
    cl = c_out // 16

    @functools.partial(
        pl.kernel, mesh=mesh,
        out_type=jax.ShapeDtypeStruct((n_rows, c_out), jnp.float32),
        scratch_types=[
            pltpu.VMEM((chunks_per_w, rows_per_chunk), jnp.int32),
            pltpu.VMEM((chunks_per_w * rows_per_chunk // 8, 128), jnp.float32),
            pltpu.VMEM((rows_per_chunk, c_out), jnp.float32),
            pltpu.VMEM((rows_per_chunk, c_out), jnp.float32),
            pltpu.VMEM((ppc, c_out), jnp.float32),
            pltpu.VMEM((ppc, c_out), jnp.float32),
            pltpu.SemaphoreType.DMA,
            pltpu.SemaphoreType.DMA,
            pltpu.SemaphoreType.DMA,
            pltpu.SemaphoreType.DMA,
        ],
    )
    def k(p1t_hbm, gidx_hbm, gw_hbm, out_hbm,
          idx_all, w_all, rows_a, rows_b, out_a, out_b, gs_a, gs_b, os_a, os_b):
        wid = lax.axis_index("s") * nc + lax.axis_index("c")
        rows_v = (rows_a, rows_b)
        out_v = (out_a, out_b)
        gsem = (gs_a, gs_b)
        osem = (os_a, os_b)

        w_rows = chunks_per_w * rows_per_chunk // 8
        pltpu.sync_copy(gidx_hbm.at[pl.ds(wid * chunks_per_w, chunks_per_w)],
                        idx_all)
        pltpu.sync_copy(gw_hbm.at[pl.ds(wid * w_rows, w_rows)], w_all)

        pltpu.async_copy(p1t_hbm.at[idx_all.at[0]], rows_v[0], gsem[0])
        pltpu.async_copy(p1t_hbm.at[idx_all.at[1]], rows_v[1], gsem[1])

        @pl.loop(0, chunks_per_w, step=2)
        def chunk_loop(g):
            for bsel in range(2):
                j = g + bsel
                rv = rows_v[bsel]
                ov = out_v[bsel]
                pltpu.make_async_copy(p1t_hbm.at[idx_all.at[j]], rv,
                                      gsem[bsel]).wait()

                @pl.when(g > 0)
                def _(ov=ov, bsel=bsel):
                    pltpu.make_async_copy(
                        ov, out_hbm.at[pl.ds(0, ppc)], osem[bsel]).wait()

                wbase = j * rows_per_chunk

                @plsc.parallel_loop(0, ppc, unroll=2)
                def point_body(p, rv=rv, ov=ov, wbase=wbase):
                    q = p * 3
                    wk = []
                    for k0 in range(3):
                        f = wbase + q + k0
                        wk.append(w_all[f // 8, pl.ds((f % 8) * 16, 16)])
                    for c in range(cl):
                        s = pl.ds(c * 16, 16)
                        acc = (rv[q, s] * wk[0]
                               + rv[q + 1, s] * wk[1]
                               + rv[q + 2, s] * wk[2])
                        ov[p, s] = acc

                pltpu.async_copy(
                    ov, out_hbm.at[pl.ds((wid * chunks_per_w + j) * ppc, ppc)],
                    osem[bsel])

                @pl.when(j + 2 < chunks_per_w)
                def _(rv=rv, bsel=bsel, j=j):
                    pltpu.async_copy(p1t_hbm.at[idx_all.at[j + 2]], rv,
                                     gsem[bsel])

        for bsel in range(2):
            pltpu.make_async_copy(out_v[bsel], out_hbm.at[pl.ds(0, ppc)],
                                  osem[bsel]).wait()

    return k(p1t_flat, gidx, gw)


def kernel(xyz_1, xyz_2, points_1, points_2, W1, b1, W2, b2):
    b, _, m = xyz_1.shape
    n = xyz_2.shape[2]
    c_in = points_1.shape[1]
    c_out = points_2.shape[1]
    tn = min(_TN, n)
    nt = n // tn
    nh = 2 if b % 2 == 0 else 1
    bh = b // nh

    b1r = b1[None, :]
    b2c = b2[:, None]

    p1t_flat = pl.pallas_call(
        _p1t_body,
        grid=(b,),
        in_specs=[
            pl.BlockSpec((1, c_in, m), lambda bi: (bi, 0, 0)),
            pl.BlockSpec((c_out, c_in), lambda bi: (0, 0)),
            pl.BlockSpec((1, c_out), lambda bi: (0, 0)),
        ],
        out_specs=pl.BlockSpec((m, c_out), lambda bi: (bi, 0)),
        out_shape=jax.ShapeDtypeStruct((b * m, c_out), jnp.float32),
    )(points_1, W1, b1r)

    out = None
    for h in range(nh):
        b0 = h * bh
        idx3, w3 = pl.pallas_call(
            functools.partial(_knn_body, m=m, k_nn=3, batch0=b0),
            grid=(bh, nt),
            in_specs=[
                pl.BlockSpec((1, 3, m), lambda bi, ti, b0=b0: (bi + b0, 0, 0)),
                pl.BlockSpec((1, 3, tn), lambda bi, ti, b0=b0: (bi + b0, 0, ti)),
            ],
            out_specs=[
                pl.BlockSpec((1, tn, 3), lambda bi, ti: (bi, ti, 0)),
                pl.BlockSpec((1, tn, 48), lambda bi, ti: (bi, ti, 0)),
            ],
            out_shape=[
                jax.ShapeDtypeStruct((bh, n, 3), jnp.int32),
                jax.ShapeDtypeStruct((bh, n, 48), jnp.float32),
            ],
        )(xyz_1, xyz_2)

        n_rows = bh * n
        gidx = idx3.reshape(n_rows // _PTS_PER_CHUNK, _PTS_PER_CHUNK * 3)
        gw = w3.reshape(n_rows * 3 // 8, 128)

        interp = _sc_gather(p1t_flat, gidx, gw, n_rows, c_out)

        in_specs = [
            pl.BlockSpec((1, c_out, tn), lambda bi, ti, b0=b0: (bi + b0, 0, ti)),
            pl.BlockSpec((c_out, c_out), lambda bi, ti: (0, 0)),
            pl.BlockSpec((c_out, 1), lambda bi, ti: (0, 0)),
            pl.BlockSpec((tn, c_out), lambda bi, ti: (bi * nt + ti, 0)),
        ]
        out_spec = pl.BlockSpec((1, c_out, tn),
                                lambda bi, ti, b0=b0: (bi + b0, 0, ti))
        out_shape = jax.ShapeDtypeStruct((b, c_out, n), jnp.float32)
        if h == 0:
            out = pl.pallas_call(
                _interp_tc_body, grid=(bh, nt), in_specs=in_specs,
                out_specs=out_spec, out_shape=out_shape,
            )(points_2, W2, b2c, interp)
        else:
            out = pl.pallas_call(
                _interp_tc_body2, grid=(bh, nt),
                in_specs=in_specs + [pl.BlockSpec(memory_space=pl.ANY)],
                out_specs=out_spec, out_shape=out_shape,
                input_output_aliases={4: 0},
            )(points_2, W2, b2c, interp, out)
    return (xyz_2, out)


# speedup vs baseline: 1.0778x; 1.0023x over previous
import functools

import jax
import jax.numpy as jnp
from jax import lax
from jax.experimental import pallas as pl
from jax.experimental.pallas import tpu as pltpu
from jax.experimental.pallas import tpu_sc as plsc

_TN = 4096
_PTS_PER_CHUNK = 32


def _p1t_body(p1_ref, w1_ref, b1r_ref, out_ref):
    dn = (((0,), (1,)), ((), ()))
    out_ref[...] = lax.dot_general(p1_ref[0], w1_ref[...], dn,
                                   preferred_element_type=jnp.float32) + b1r_ref[...]


def _knn_body(xyz1_ref, xyz2_ref, idx_ref, w_ref, *, m, k_nn, batch0):
    x1 = xyz1_ref[0]
    x2 = xyz2_ref[0].T

    d = None
    for c in range(x1.shape[0]):
        diff = x2[:, c:c + 1] - x1[c:c + 1, :]
        sq = diff * diff
        d = sq if d is None else d + sq

    iota = lax.broadcasted_iota(jnp.int32, d.shape, 1)
    big = jnp.float32(3.0e38)

    d_ks, i_ks = [], []
    dd = d
    for _ in range(k_nn):
        dmin = jnp.min(dd, axis=1, keepdims=True)
        sel = dd == dmin
        idx = jnp.min(jnp.where(sel, iota, jnp.int32(m)), axis=1,
                      keepdims=True)
        d_ks.append(dmin)
        i_ks.append(idx)
        dd = jnp.where(iota == idx, big, dd)

    recips = [1.0 / (dk + jnp.float32(1e-8)) for dk in d_ks]
    norm = functools.reduce(lambda a, b: a + b, recips)
    weights = [r / norm for r in recips]

    base = (batch0 + pl.program_id(0)) * m
    idx_ref[0] = jnp.concatenate(i_ks, axis=1) + base
    w_ref[0] = jnp.concatenate(
        [jnp.broadcast_to(wk, (wk.shape[0], 16)) for wk in weights], axis=1)


def _interp_tc_body(p2_ref, w2_ref, b2_ref, interp_ref, out_ref):
    dn = (((1,), (0,)), ((), ()))
    p2 = lax.dot_general(w2_ref[...], p2_ref[0], dn,
                         preferred_element_type=jnp.float32) + b2_ref[...]
    out_ref[0] = p2 + interp_ref[...].T


def _interp_tc_body2(p2_ref, w2_ref, b2_ref, interp_ref, prev_ref, out_ref):
    del prev_ref
    _interp_tc_body(p2_ref, w2_ref, b2_ref, interp_ref, out_ref)


def _sc_gather(p1t_flat, gidx, gw, n_rows, c_out):
    info = plsc.get_sparse_core_info()
    nc, ns = info.num_cores, info.num_subcores
    nw = nc * ns
    ppc = _PTS_PER_CHUNK
    rows_per_chunk = ppc * 3
    chunks_total = n_rows // ppc
    chunks_per_w = chunks_total // nw
    mesh = plsc.VectorSubcoreMesh(core_axis_name="c", subcore_axis_name="s")
    cl = c_out // 16

    @functools.partial(
        pl.kernel, mesh=mesh,
        out_type=jax.ShapeDtypeStruct((n_rows, c_out), jnp.float32),
        scratch_types=[
            pltpu.VMEM((chunks_per_w, rows_per_chunk), jnp.int32),
            pltpu.VMEM((chunks_per_w * rows_per_chunk // 8, 128), jnp.float32),
            pltpu.VMEM((rows_per_chunk, c_out), jnp.float32),
            pltpu.VMEM((rows_per_chunk, c_out), jnp.float32),
            pltpu.VMEM((ppc, c_out), jnp.float32),
            pltpu.VMEM((ppc, c_out), jnp.float32),
            pltpu.SemaphoreType.DMA,
            pltpu.SemaphoreType.DMA,
            pltpu.SemaphoreType.DMA,
            pltpu.SemaphoreType.DMA,
        ],
    )
    def k(p1t_hbm, gidx_hbm, gw_hbm, out_hbm,
          idx_all, w_all, rows_a, rows_b, out_a, out_b, gs_a, gs_b, os_a, os_b):
        wid = lax.axis_index("s") * nc + lax.axis_index("c")
        rows_v = (rows_a, rows_b)
        out_v = (out_a, out_b)
        gsem = (gs_a, gs_b)
        osem = (os_a, os_b)

        w_rows = chunks_per_w * rows_per_chunk // 8
        pltpu.sync_copy(gidx_hbm.at[pl.ds(wid * chunks_per_w, chunks_per_w)],
                        idx_all)
        pltpu.sync_copy(gw_hbm.at[pl.ds(wid * w_rows, w_rows)], w_all)

        pltpu.async_copy(p1t_hbm.at[idx_all.at[0]], rows_v[0], gsem[0])
        pltpu.async_copy(p1t_hbm.at[idx_all.at[1]], rows_v[1], gsem[1])

        @pl.loop(0, chunks_per_w, step=2)
        def chunk_loop(g):
            for bsel in range(2):
                j = g + bsel
                rv = rows_v[bsel]
                ov = out_v[bsel]
                pltpu.make_async_copy(p1t_hbm.at[idx_all.at[j]], rv,
                                      gsem[bsel]).wait()

                @pl.when(g > 0)
                def _(ov=ov, bsel=bsel):
                    pltpu.make_async_copy(
                        ov, out_hbm.at[pl.ds(0, ppc)], osem[bsel]).wait()

                wbase = j * rows_per_chunk

                @plsc.parallel_loop(0, ppc, unroll=2)
                def point_body(p, rv=rv, ov=ov, wbase=wbase):
                    q = p * 3
                    wk = []
                    for k0 in range(3):
                        f = wbase + q + k0
                        wk.append(w_all[f // 8, pl.ds((f % 8) * 16, 16)])
                    for c in range(cl):
                        s = pl.ds(c * 16, 16)
                        acc = (rv[q, s] * wk[0]
                               + rv[q + 1, s] * wk[1]
                               + rv[q + 2, s] * wk[2])
                        ov[p, s] = acc

                pltpu.async_copy(
                    ov, out_hbm.at[pl.ds((wid * chunks_per_w + j) * ppc, ppc)],
                    osem[bsel])

                @pl.when(j + 2 < chunks_per_w)
                def _(rv=rv, bsel=bsel, j=j):
                    pltpu.async_copy(p1t_hbm.at[idx_all.at[j + 2]], rv,
                                     gsem[bsel])

        for bsel in range(2):
            pltpu.make_async_copy(out_v[bsel], out_hbm.at[pl.ds(0, ppc)],
                                  osem[bsel]).wait()

    return k(p1t_flat, gidx, gw)


def kernel(xyz_1, xyz_2, points_1, points_2, W1, b1, W2, b2):
    b, _, m = xyz_1.shape
    n = xyz_2.shape[2]
    c_in = points_1.shape[1]
    c_out = points_2.shape[1]
    tn = min(_TN, n)
    nt = n // tn
    nh = 2 if b % 2 == 0 else 1
    bh = b // nh

    b1r = b1[None, :]
    b2c = b2[:, None]

    p1t_flat = pl.pallas_call(
        _p1t_body,
        grid=(b,),
        in_specs=[
            pl.BlockSpec((1, c_in, m), lambda bi: (bi, 0, 0)),
            pl.BlockSpec((c_out, c_in), lambda bi: (0, 0)),
            pl.BlockSpec((1, c_out), lambda bi: (0, 0)),
        ],
        out_specs=pl.BlockSpec((m, c_out), lambda bi: (bi, 0)),
        out_shape=jax.ShapeDtypeStruct((b * m, c_out), jnp.float32),
    )(points_1, W1, b1r)

    out = None
    for h in range(nh):
        b0 = h * bh
        idx3, w3 = pl.pallas_call(
            functools.partial(_knn_body, m=m, k_nn=3, batch0=b0),
            grid=(bh, nt),
            in_specs=[
                pl.BlockSpec((1, 3, m), lambda bi, ti, b0=b0: (bi + b0, 0, 0)),
                pl.BlockSpec((1, 3, tn), lambda bi, ti, b0=b0: (bi + b0, 0, ti)),
            ],
            out_specs=[
                pl.BlockSpec((1, tn, 3), lambda bi, ti: (bi, ti, 0)),
                pl.BlockSpec((1, tn, 48), lambda bi, ti: (bi, ti, 0)),
            ],
            out_shape=[
                jax.ShapeDtypeStruct((bh, n, 3), jnp.int32),
                jax.ShapeDtypeStruct((bh, n, 48), jnp.float32),
            ],
        )(xyz_1, xyz_2)

        n_rows = bh * n
        gidx = idx3.reshape(n_rows // _PTS_PER_CHUNK, _PTS_PER_CHUNK * 3)
        gw = w3.reshape(n_rows * 3 // 8, 128)

        interp = _sc_gather(p1t_flat, gidx, gw, n_rows, c_out)

        in_specs = [
            pl.BlockSpec((1, c_out, tn), lambda bi, ti, b0=b0: (bi + b0, 0, ti)),
            pl.BlockSpec((c_out, c_out), lambda bi, ti: (0, 0)),
            pl.BlockSpec((c_out, 1), lambda bi, ti: (0, 0)),
            pl.BlockSpec((tn, c_out), lambda bi, ti: (bi * nt + ti, 0)),
        ]
        out_spec = pl.BlockSpec((1, c_out, tn),
                                lambda bi, ti, b0=b0: (bi + b0, 0, ti))
        out_shape = jax.ShapeDtypeStruct((b, c_out, n), jnp.float32)
        if h == 0:
            out = pl.pallas_call(
                _interp_tc_body, grid=(bh, nt), in_specs=in_specs,
                out_specs=out_spec, out_shape=out_shape,
            )(points_2, W2, b2c, interp)
        else:
            out = pl.pallas_call(
                _interp_tc_body2, grid=(bh, nt),
                in_specs=in_specs + [pl.BlockSpec(memory_space=pl.ANY)],
                out_specs=out_spec, out_shape=out_shape,
                input_output_aliases={4: 0},
            )(points_2, W2, b2c, interp, out)
    return (xyz_2, out)


# speedup vs baseline: 1.0791x; 1.0012x over previous
import functools

import jax
import jax.numpy as jnp
from jax import lax
from jax.experimental import pallas as pl
from jax.experimental.pallas import tpu as pltpu
from jax.experimental.pallas import tpu_sc as plsc

_TN = 4096
_PTS_PER_CHUNK = 32


def _p1t_body(p1_ref, w1_ref, b1r_ref, out_ref):
    dn = (((0,), (1,)), ((), ()))
    out_ref[...] = lax.dot_general(p1_ref[0], w1_ref[...], dn,
                                   preferred_element_type=jnp.float32) + b1r_ref[...]


def _knn_body(xyz1_ref, xyz2_ref, idx_ref, w_ref, *, m, k_nn, batch0):
    x1 = xyz1_ref[0]
    x2 = xyz2_ref[0].T

    d = None
    for c in range(x1.shape[0]):
        diff = x2[:, c:c + 1] - x1[c:c + 1, :]
        sq = diff * diff
        d = sq if d is None else d + sq

    iota = lax.broadcasted_iota(jnp.int32, d.shape, 1)
    big = jnp.float32(3.0e38)

    d_ks, i_ks = [], []
    dd = d
    for r in range(k_nn):
        dmin = jnp.min(dd, axis=1, keepdims=True)
        sel = dd == dmin
        idx = jnp.min(jnp.where(sel, iota, jnp.int32(m)), axis=1,
                      keepdims=True)
        d_ks.append(dmin)
        i_ks.append(idx)
        if r + 1 < k_nn:
            dd = jnp.where(iota == idx, big, dd)

    recips = [1.0 / (dk + jnp.float32(1e-8)) for dk in d_ks]
    norm = functools.reduce(lambda a, b: a + b, recips)
    weights = [r / norm for r in recips]

    base = (batch0 + pl.program_id(0)) * m
    idx_ref[0] = jnp.concatenate(i_ks, axis=1) + base
    w_ref[0] = jnp.concatenate(
        [jnp.broadcast_to(wk, (wk.shape[0], 16)) for wk in weights], axis=1)


def _interp_tc_body(p2_ref, w2_ref, b2_ref, interp_ref, out_ref):
    dn = (((1,), (0,)), ((), ()))
    p2 = lax.dot_general(w2_ref[...], p2_ref[0], dn,
                         preferred_element_type=jnp.float32) + b2_ref[...]
    out_ref[0] = p2 + interp_ref[...].T


def _interp_tc_body2(p2_ref, w2_ref, b2_ref, interp_ref, prev_ref, out_ref):
    del prev_ref
    _interp_tc_body(p2_ref, w2_ref, b2_ref, interp_ref, out_ref)


def _sc_gather(p1t_flat, gidx, gw, n_rows, c_out):
    info = plsc.get_sparse_core_info()
    nc, ns = info.num_cores, info.num_subcores
    nw = nc * ns
    ppc = _PTS_PER_CHUNK
    rows_per_chunk = ppc * 3
    chunks_total = n_rows // ppc
    chunks_per_w = chunks_total // nw
    mesh = plsc.VectorSubcoreMesh(core_axis_name="c", subcore_axis_name="s")
    cl = c_out // 16

    @functools.partial(
        pl.kernel, mesh=mesh,
        out_type=jax.ShapeDtypeStruct((n_rows, c_out), jnp.float32),
        scratch_types=[
            pltpu.VMEM((chunks_per_w, rows_per_chunk), jnp.int32),
            pltpu.VMEM((chunks_per_w * rows_per_chunk // 8, 128), jnp.float32),
            pltpu.VMEM((rows_per_chunk, c_out), jnp.float32),
            pltpu.VMEM((rows_per_chunk, c_out), jnp.float32),
            pltpu.VMEM((ppc, c_out), jnp.float32),
            pltpu.VMEM((ppc, c_out), jnp.float32),
            pltpu.SemaphoreType.DMA,
            pltpu.SemaphoreType.DMA,
            pltpu.SemaphoreType.DMA,
            pltpu.SemaphoreType.DMA,
        ],
    )
    def k(p1t_hbm, gidx_hbm, gw_hbm, out_hbm,
          idx_all, w_all, rows_a, rows_b, out_a, out_b, gs_a, gs_b, os_a, os_b):
        wid = lax.axis_index("s") * nc + lax.axis_index("c")
        rows_v = (rows_a, rows_b)
        out_v = (out_a, out_b)
        gsem = (gs_a, gs_b)
        osem = (os_a, os_b)

        w_rows = chunks_per_w * rows_per_chunk // 8
        pltpu.sync_copy(gidx_hbm.at[pl.ds(wid * chunks_per_w, chunks_per_w)],
                        idx_all)
        pltpu.sync_copy(gw_hbm.at[pl.ds(wid * w_rows, w_rows)], w_all)

        pltpu.async_copy(p1t_hbm.at[idx_all.at[0]], rows_v[0], gsem[0])
        pltpu.async_copy(p1t_hbm.at[idx_all.at[1]], rows_v[1], gsem[1])

        @pl.loop(0, chunks_per_w, step=2)
        def chunk_loop(g):
            for bsel in range(2):
                j = g + bsel
                rv = rows_v[bsel]
                ov = out_v[bsel]
                pltpu.make_async_copy(p1t_hbm.at[idx_all.at[j]], rv,
                                      gsem[bsel]).wait()

                @pl.when(g > 0)
                def _(ov=ov, bsel=bsel):
                    pltpu.make_async_copy(
                        ov, out_hbm.at[pl.ds(0, ppc)], osem[bsel]).wait()

                wbase = j * rows_per_chunk

                @plsc.parallel_loop(0, ppc, unroll=2)
                def point_body(p, rv=rv, ov=ov, wbase=wbase):
                    q = p * 3
                    wk = []
                    for k0 in range(3):
                        f = wbase + q + k0
                        wk.append(w_all[f // 8, pl.ds((f % 8) * 16, 16)])
                    for c in range(cl):
                        s = pl.ds(c * 16, 16)
                        acc = (rv[q, s] * wk[0]
                               + rv[q + 1, s] * wk[1]
                               + rv[q + 2, s] * wk[2])
                        ov[p, s] = acc

                pltpu.async_copy(
                    ov, out_hbm.at[pl.ds((wid * chunks_per_w + j) * ppc, ppc)],
                    osem[bsel])

                @pl.when(j + 2 < chunks_per_w)
                def _(rv=rv, bsel=bsel, j=j):
                    pltpu.async_copy(p1t_hbm.at[idx_all.at[j + 2]], rv,
                                     gsem[bsel])

        for bsel in range(2):
            pltpu.make_async_copy(out_v[bsel], out_hbm.at[pl.ds(0, ppc)],
                                  osem[bsel]).wait()

    return k(p1t_flat, gidx, gw)


def kernel(xyz_1, xyz_2, points_1, points_2, W1, b1, W2, b2):
    b, _, m = xyz_1.shape
    n = xyz_2.shape[2]
    c_in = points_1.shape[1]
    c_out = points_2.shape[1]
    tn = min(_TN, n)
    nt = n // tn
    nh = 2 if b % 2 == 0 else 1
    bh = b // nh

    b1r = b1[None, :]
    b2c = b2[:, None]

    p1t_flat = pl.pallas_call(
        _p1t_body,
        grid=(b,),
        in_specs=[
            pl.BlockSpec((1, c_in, m), lambda bi: (bi, 0, 0)),
            pl.BlockSpec((c_out, c_in), lambda bi: (0, 0)),
            pl.BlockSpec((1, c_out), lambda bi: (0, 0)),
        ],
        out_specs=pl.BlockSpec((m, c_out), lambda bi: (bi, 0)),
        out_shape=jax.ShapeDtypeStruct((b * m, c_out), jnp.float32),
    )(points_1, W1, b1r)

    out = None
    for h in range(nh):
        b0 = h * bh
        idx3, w3 = pl.pallas_call(
            functools.partial(_knn_body, m=m, k_nn=3, batch0=b0),
            grid=(bh, nt),
            in_specs=[
                pl.BlockSpec((1, 3, m), lambda bi, ti, b0=b0: (bi + b0, 0, 0)),
                pl.BlockSpec((1, 3, tn), lambda bi, ti, b0=b0: (bi + b0, 0, ti)),
            ],
            out_specs=[
                pl.BlockSpec((1, tn, 3), lambda bi, ti: (bi, ti, 0)),
                pl.BlockSpec((1, tn, 48), lambda bi, ti: (bi, ti, 0)),
            ],
            out_shape=[
                jax.ShapeDtypeStruct((bh, n, 3), jnp.int32),
                jax.ShapeDtypeStruct((bh, n, 48), jnp.float32),
            ],
        )(xyz_1, xyz_2)

        n_rows = bh * n
        gidx = idx3.reshape(n_rows // _PTS_PER_CHUNK, _PTS_PER_CHUNK * 3)
        gw = w3.reshape(n_rows * 3 // 8, 128)

        interp = _sc_gather(p1t_flat, gidx, gw, n_rows, c_out)

        in_specs = [
            pl.BlockSpec((1, c_out, tn), lambda bi, ti, b0=b0: (bi + b0, 0, ti)),
            pl.BlockSpec((c_out, c_out), lambda bi, ti: (0, 0)),
            pl.BlockSpec((c_out, 1), lambda bi, ti: (0, 0)),
            pl.BlockSpec((tn, c_out), lambda bi, ti: (bi * nt + ti, 0)),
        ]
        out_spec = pl.BlockSpec((1, c_out, tn),
                                lambda bi, ti, b0=b0: (bi + b0, 0, ti))
        out_shape = jax.ShapeDtypeStruct((b, c_out, n), jnp.float32)
        if h == 0:
            out = pl.pallas_call(
                _interp_tc_body, grid=(bh, nt), in_specs=in_specs,
                out_specs=out_spec, out_shape=out_shape,
            )(points_2, W2, b2c, interp)
        else:
            out = pl.pallas_call(
                _interp_tc_body2, grid=(bh, nt),
                in_specs=in_specs + [pl.BlockSpec(memory_space=pl.ANY)],
                out_specs=out_spec, out_shape=out_shape,
                input_output_aliases={4: 0},
            )(points_2, W2, b2c, interp, out)
    return (xyz_2, out)
